# baseline (speedup 1.0000x reference)
import functools
import math

import jax
import jax.numpy as jnp
from jax import lax
from jax.experimental import pallas as pl
from jax.experimental.pallas import tpu as pltpu
import jax.experimental.pallas.tpu_sc as plsc

_D = 768
_E = 8
_K = 2
_H = 2048
_N = 4096
_NK = _N * _K
_CAP = 1280
_EC = _E * _CAP
_AUX_W = 0.01
_Z_W = 0.001

_SC_NC = 2
_SC_NS = 16
_NW = _SC_NC * _SC_NS

_BT = 256
_NTB = _N // _BT

_CB = 256
_BH = 512

_SC_SCAT_CH = _NK // _NW // 64
_SC_GATH_ROWS = _NK // _NW


def _router_body(x_ref, wg_ref, idx_ref, dst0_ref, dst1_ref, g0_ref, g1_ref,
                 w0_ref, w1_ref, aux_ref, base_ref, pm_ref, fr_ref, z2_ref):
    tb = pl.program_id(0)

    @pl.when(tb == 0)
    def _init():
        base_ref[...] = jnp.zeros_like(base_ref)
        pm_ref[...] = jnp.zeros_like(pm_ref)
        fr_ref[...] = jnp.zeros_like(fr_ref)
        z2_ref[...] = jnp.zeros_like(z2_ref)

    x = x_ref[...]
    wg = wg_ref[...]
    logits = lax.dot_general(x, wg, (((1,), (1,)), ((), ())),
                             preferred_element_type=jnp.float32)

    io_e = lax.broadcasted_iota(jnp.int32, (_BT, _E), 1).astype(jnp.float32)
    m1 = jnp.max(logits, axis=1, keepdims=True)
    idx1 = jnp.min(jnp.where(logits == m1, io_e, 1e9), axis=1, keepdims=True)
    oh1 = (io_e == idx1).astype(jnp.float32)
    neg = jnp.where(oh1 > 0, -jnp.inf, logits)
    m2 = jnp.max(neg, axis=1, keepdims=True)
    idx2 = jnp.min(jnp.where(neg == m2, io_e, 1e9), axis=1, keepdims=True)
    oh2 = (io_e == idx2).astype(jnp.float32)

    e2 = jnp.exp(m2 - m1)
    p1 = 1.0 / (1.0 + e2)
    p2 = 1.0 - p1

    ex = jnp.exp(logits - m1)
    zden = jnp.sum(ex, axis=1, keepdims=True)
    probs = ex / zden
    pm_ref[...] += jnp.sum(probs, axis=0, keepdims=True)
    fr_ref[...] += jnp.sum(oh1, axis=0, keepdims=True)
    zlse = m1 + jnp.log(zden)
    z2_ref[...] += jnp.sum(zlse * zlse, axis=(0, 1), keepdims=True)

    s = oh1 + oh2
    row_i = lax.broadcasted_iota(jnp.int32, (_BT, _BT), 0)
    col_i = lax.broadcasted_iota(jnp.int32, (_BT, _BT), 1)
    tril = (row_i > col_i).astype(jnp.float32)
    prefix = lax.dot_general(tril, s, (((1,), (0,)), ((), ())),
                             preferred_element_type=jnp.float32)
    pb = prefix + base_ref[...]
    rank0 = jnp.sum(pb * oh1, axis=1, keepdims=True)
    rank1 = (jnp.sum(pb * oh2, axis=1, keepdims=True)
             + jnp.sum(oh1 * oh2, axis=1, keepdims=True))
    base_ref[...] += jnp.sum(s, axis=0, keepdims=True)

    keep0 = (rank0 < _CAP).astype(jnp.float32)
    keep1 = (rank1 < _CAP).astype(jnp.float32)
    slot0 = jnp.minimum(rank0, _CAP - 1.0)
    slot1 = jnp.minimum(rank1, _CAP - 1.0)
    tgt0 = jnp.sum(io_e * oh1, axis=1, keepdims=True)
    tgt1 = jnp.sum(io_e * oh2, axis=1, keepdims=True)
    gidx0 = tgt0 * _CAP + slot0
    gidx1 = tgt1 * _CAP + slot1

    n_io = lax.broadcasted_iota(jnp.int32, (_BT, 1), 0).astype(jnp.float32)
    dump = _EC + jnp.mod(n_io, 8.0)
    dst0_ref[...] = jnp.where(keep0 > 0, gidx0, dump).astype(jnp.int32)
    dst1_ref[...] = jnp.where(keep1 > 0, gidx1, dump).astype(jnp.int32)
    g0_ref[...] = gidx0.astype(jnp.int32)
    g1_ref[...] = gidx1.astype(jnp.int32)

    wa = p1 * keep0
    wb = p2 * keep1
    den = jnp.maximum(wa + wb, 1e-12)
    w0_ref[...] = wa / den
    w1_ref[...] = wb / den
    idx_ref[...] = jnp.concatenate([tgt0, tgt1], axis=1).astype(jnp.int32)

    pm = pm_ref[...] / _N
    fr = fr_ref[...] / _N
    bal = jnp.sum(pm * fr, axis=(0, 1), keepdims=True) * _E
    z2 = z2_ref[...] / _N
    aux_ref[...] = _AUX_W * bal + _Z_W * z2


def _router(x_flat, w_gate):
    f32 = jnp.float32
    i32 = jnp.int32
    return pl.pallas_call(
        _router_body,
        grid=(_NTB,),
        in_specs=[
            pl.BlockSpec((_BT, _D), lambda tb: (tb, 0)),
            pl.BlockSpec((_E, _D), lambda tb: (0, 0)),
        ],
        out_specs=[
            pl.BlockSpec((_BT, _K), lambda tb: (tb, 0)),
            pl.BlockSpec((_BT, 1), lambda tb: (tb, 0)),
            pl.BlockSpec((_BT, 1), lambda tb: (tb, 0)),
            pl.BlockSpec((_BT, 1), lambda tb: (tb, 0)),
            pl.BlockSpec((_BT, 1), lambda tb: (tb, 0)),
            pl.BlockSpec((_BT, 1), lambda tb: (tb, 0)),
            pl.BlockSpec((_BT, 1), lambda tb: (tb, 0)),
            pl.BlockSpec((1, 1), lambda tb: (0, 0)),
        ],
        out_shape=[
            jax.ShapeDtypeStruct((_N, _K), i32),
            jax.ShapeDtypeStruct((_N, 1), i32),
            jax.ShapeDtypeStruct((_N, 1), i32),
            jax.ShapeDtypeStruct((_N, 1), i32),
            jax.ShapeDtypeStruct((_N, 1), i32),
            jax.ShapeDtypeStruct((_N, 1), f32),
            jax.ShapeDtypeStruct((_N, 1), f32),
            jax.ShapeDtypeStruct((1, 1), f32),
        ],
        scratch_shapes=[
            pltpu.VMEM((1, _E), f32),
            pltpu.VMEM((1, _E), f32),
            pltpu.VMEM((1, _E), f32),
            pltpu.VMEM((1, 1), f32),
        ],
    )(x_flat, w_gate)


def _sc_scatter_body(x_hbm, idx_hbm, buf_hbm, idxg_v, src_v, rows_v, sem):
    w = lax.axis_index("s") * _SC_NC + lax.axis_index("c")
    base = w * (_SC_SCAT_CH * 64)
    pltpu.sync_copy(idx_hbm.at[w], idxg_v)
    for ch in range(_SC_SCAT_CH):
        cb = base + ch * 64
        for v in range(4):
            src_v[pl.ds(v * 16, 16)] = lax.rem(
                cb + v * 16 + lax.iota(jnp.int32, 16), _N)
        pltpu.async_copy(x_hbm.at[src_v], rows_v, sem).wait()
        pltpu.async_copy(rows_v, buf_hbm.at[idxg_v.at[ch]], sem).wait()


@functools.lru_cache(maxsize=None)
def _sc_scatter_kernel():
    return pl.kernel(
        _sc_scatter_body,
        out_type=jax.ShapeDtypeStruct((_EC + 8, _D), jnp.float32),
        mesh=plsc.VectorSubcoreMesh(core_axis_name="c", subcore_axis_name="s",
                                    num_cores=_SC_NC, num_subcores=_SC_NS),
        scratch_types=[
            pltpu.VMEM((_SC_SCAT_CH, 64), jnp.int32),
            pltpu.VMEM((64,), jnp.int32),
            pltpu.VMEM((64, _D), jnp.float32),
            pltpu.SemaphoreType.DMA,
        ],
    )


def _sc_scatter(x_flat, dst3):
    return _sc_scatter_kernel()(x_flat, dst3)


def _sc_gather_body(tab_hbm, idx_hbm, out_hbm, idx_v, rows_v, sem):
    w = lax.axis_index("s") * _SC_NC + lax.axis_index("c")
    base = w * _SC_GATH_ROWS
    pltpu.sync_copy(idx_hbm.at[pl.ds(base, _SC_GATH_ROWS)], idx_v)
    for ch in range(_SC_GATH_ROWS // 64):
        pltpu.async_copy(tab_hbm.at[idx_v.at[pl.ds(ch * 64, 64)]],
                         rows_v, sem).wait()
        pltpu.sync_copy(rows_v, out_hbm.at[pl.ds(base + ch * 64, 64)])


@functools.lru_cache(maxsize=None)
def _sc_gather_kernel():
    return pl.kernel(
        _sc_gather_body,
        out_type=jax.ShapeDtypeStruct((_NK, _D), jnp.float32),
        mesh=plsc.VectorSubcoreMesh(core_axis_name="c", subcore_axis_name="s",
                                    num_cores=_SC_NC, num_subcores=_SC_NS),
        scratch_types=[
            pltpu.VMEM((_SC_GATH_ROWS,), jnp.int32),
            pltpu.VMEM((64, _D), jnp.float32),
            pltpu.SemaphoreType.DMA,
        ],
    )


def _sc_gather(tab, gidx):
    return _sc_gather_kernel()(tab, gidx)


def _ffn_body(buf_ref, wg_ref, wu_ref, w2_ref, out_ref):
    hb = pl.program_id(2)
    bf16 = jnp.bfloat16
    xb = buf_ref[...].astype(bf16)
    g = jnp.dot(xb, wg_ref[0].astype(bf16), preferred_element_type=jnp.float32)
    u = jnp.dot(xb, wu_ref[0].astype(bf16), preferred_element_type=jnp.float32)
    act = g * (1.0 / (1.0 + jnp.exp(-g))) * u
    part = jnp.dot(act.astype(bf16), w2_ref[0].astype(bf16),
                   preferred_element_type=jnp.float32)

    @pl.when(hb == 0)
    def _set():
        out_ref[...] = part

    @pl.when(hb > 0)
    def _acc():
        out_ref[...] += part


def _ffn(buf, w13, w2):
    ncb = _CAP // _CB
    nhb = _H // _BH
    return pl.pallas_call(
        _ffn_body,
        grid=(_E, ncb, nhb),
        in_specs=[
            pl.BlockSpec((_CB, _D), lambda e, c, h: (e * ncb + c, 0)),
            pl.BlockSpec((1, _D, _BH), lambda e, c, h: (e, 0, h)),
            pl.BlockSpec((1, _D, _BH), lambda e, c, h: (e, 0, nhb + h)),
            pl.BlockSpec((1, _BH, _D), lambda e, c, h: (e, h, 0)),
        ],
        out_specs=pl.BlockSpec((_CB, _D), lambda e, c, h: (e * ncb + c, 0)),
        out_shape=jax.ShapeDtypeStruct((_EC, _D), jnp.float32),
        compiler_params=pltpu.CompilerParams(
            dimension_semantics=("arbitrary", "arbitrary", "arbitrary")),
    )(buf, w13, w13, w2)


def _gelu(v):
    return 0.5 * v * (1.0 + lax.erf(v * 0.7071067811865476))


def _collab_body(s0_ref, s1_ref, w0_ref, w1_ref, wm_ref, wq_ref, wk_ref,
                 wu1_ref, wu2_ref, y_ref):
    w0 = w0_ref[...]
    w1 = w1_ref[...]
    km0 = (w0 > 0).astype(jnp.float32)
    km1 = (w1 > 0).astype(jnp.float32)
    s0 = jnp.where(w0 > 0, s0_ref[...], 0.0)
    s1 = jnp.where(w1 > 0, s1_ref[...], 0.0)

    bf16 = jnp.bfloat16
    wm = wm_ref[...].astype(bf16)
    wq = wq_ref[...].astype(bf16)
    wk = wk_ref[...].astype(bf16)
    s0h = s0.astype(bf16)
    s1h = s1.astype(bf16)
    m0 = jnp.dot(s0h, wm, preferred_element_type=jnp.float32)
    m1 = jnp.dot(s1h, wm, preferred_element_type=jnp.float32)
    q0 = jnp.dot(s0h, wq, preferred_element_type=jnp.float32)
    q1 = jnp.dot(s1h, wq, preferred_element_type=jnp.float32)
    k0 = jnp.dot(m0.astype(bf16), wk, preferred_element_type=jnp.float32)
    k1 = jnp.dot(m1.astype(bf16), wk, preferred_element_type=jnp.float32)

    inv = 1.0 / math.sqrt(_D)
    s00 = jnp.sum(q0 * k0, axis=1, keepdims=True) * inv
    s01 = jnp.sum(q0 * k1, axis=1, keepdims=True) * inv
    s10 = jnp.sum(q1 * k0, axis=1, keepdims=True) * inv
    s11 = jnp.sum(q1 * k1, axis=1, keepdims=True) * inv

    v00 = km0 * km0
    v01 = km0 * km1
    v10 = km1 * km0
    v11 = km1 * km1
    big = -1e9
    t00 = jnp.where(v00 > 0, s00, big)
    t01 = jnp.where(v01 > 0, s01, big)
    t10 = jnp.where(v10 > 0, s10, big)
    t11 = jnp.where(v11 > 0, s11, big)

    mx0 = jnp.maximum(t00, t01)
    p00 = jnp.exp(t00 - mx0) * v00
    p01 = jnp.exp(t01 - mx0) * v01
    z0 = jnp.maximum(p00 + p01, 1e-12)
    a00 = p00 / z0
    a01 = p01 / z0
    mx1 = jnp.maximum(t10, t11)
    p10 = jnp.exp(t10 - mx1) * v10
    p11 = jnp.exp(t11 - mx1) * v11
    z1 = jnp.maximum(p10 + p11, 1e-12)
    a10 = p10 / z1
    a11 = p11 / z1

    c0 = a00 * m0 + a01 * m1
    c1 = a10 * m0 + a11 * m1
    cat0 = jnp.concatenate([s0h, c0.astype(bf16)], axis=1)
    cat1 = jnp.concatenate([s1h, c1.astype(bf16)], axis=1)
    wu1 = wu1_ref[...].astype(bf16)
    wu2 = wu2_ref[...].astype(bf16)
    h0 = _gelu(jnp.dot(cat0, wu1, preferred_element_type=jnp.float32))
    h1 = _gelu(jnp.dot(cat1, wu1, preferred_element_type=jnp.float32))
    r0 = s0 + jnp.dot(h0.astype(bf16), wu2, preferred_element_type=jnp.float32)
    r1 = s1 + jnp.dot(h1.astype(bf16), wu2, preferred_element_type=jnp.float32)
    y_ref[...] = w0 * r0 + w1 * r1


def _collab(sel, w0, w1, wm_t, wq_t, wk_t, wu1_t, wu2_t):
    return pl.pallas_call(
        _collab_body,
        grid=(_NTB,),
        in_specs=[
            pl.BlockSpec((_BT, _D), lambda tb: (tb, 0)),
            pl.BlockSpec((_BT, _D), lambda tb: (_NTB + tb, 0)),
            pl.BlockSpec((_BT, 1), lambda tb: (tb, 0)),
            pl.BlockSpec((_BT, 1), lambda tb: (tb, 0)),
            pl.BlockSpec((_D, _D), lambda tb: (0, 0)),
            pl.BlockSpec((_D, _D), lambda tb: (0, 0)),
            pl.BlockSpec((_D, _D), lambda tb: (0, 0)),
            pl.BlockSpec((2 * _D, 2 * _D), lambda tb: (0, 0)),
            pl.BlockSpec((2 * _D, _D), lambda tb: (0, 0)),
        ],
        out_specs=pl.BlockSpec((_BT, _D), lambda tb: (tb, 0)),
        out_shape=jax.ShapeDtypeStruct((_N, _D), jnp.float32),
    )(sel, sel, w0, w1, wm_t, wq_t, wk_t, wu1_t, wu2_t)


def kernel(x, W_gate, W13, W2, Wmsg, Wq, Wk, Wu1, Wu2):
    bx, tx, dx = x.shape
    x_flat = x.reshape(-1, dx).astype(jnp.float32)

    (topk_idx, dst0, dst1, g0, g1, w0, w1, aux) = _router(x_flat, W_gate)

    dst = jnp.concatenate([dst0[:, 0], dst1[:, 0]])
    buf = _sc_scatter(x_flat, dst.reshape(_NW, _SC_SCAT_CH, 64))
    out_buf = _ffn(buf, W13, W2)
    gidx = jnp.concatenate([g0[:, 0], g1[:, 0]])
    sel = _sc_gather(out_buf, gidx)

    y = _collab(sel, w0, w1, Wmsg.T, Wq.T, Wk.T, Wu1.T, Wu2.T)
    return (y.reshape(bx, tx, dx), aux[0, 0],
            topk_idx.reshape(bx, tx, _K))


# speedup vs baseline: 1.5350x; 1.5350x over previous
import functools
import math

import jax
import jax.numpy as jnp
from jax import lax
from jax.experimental import pallas as pl
from jax.experimental.pallas import tpu as pltpu
import jax.experimental.pallas.tpu_sc as plsc

_D = 768
_E = 8
_K = 2
_H = 2048
_N = 4096
_NK = _N * _K
_CAP = 1280
_EC = _E * _CAP
_AUX_W = 0.01
_Z_W = 0.001

_SC_NC = 2
_SC_NS = 16
_NW = _SC_NC * _SC_NS

_BT = 256
_NTB = _N // _BT

_CB = 256
_BH = 512

_SC_SCAT_CH = _NK // _NW // 64
_SC_GATH_ROWS = _NK // _NW


def _router_body(x_ref, wg_ref, idx_ref, dst0_ref, dst1_ref, g0_ref, g1_ref,
                 w0_ref, w1_ref, aux_ref, base_ref, pm_ref, fr_ref, z2_ref):
    tb = pl.program_id(0)

    @pl.when(tb == 0)
    def _init():
        base_ref[...] = jnp.zeros_like(base_ref)
        pm_ref[...] = jnp.zeros_like(pm_ref)
        fr_ref[...] = jnp.zeros_like(fr_ref)
        z2_ref[...] = jnp.zeros_like(z2_ref)

    x = x_ref[...]
    wg = wg_ref[...]
    logits = lax.dot_general(x, wg, (((1,), (1,)), ((), ())),
                             preferred_element_type=jnp.float32)

    io_e = lax.broadcasted_iota(jnp.int32, (_BT, _E), 1).astype(jnp.float32)
    m1 = jnp.max(logits, axis=1, keepdims=True)
    idx1 = jnp.min(jnp.where(logits == m1, io_e, 1e9), axis=1, keepdims=True)
    oh1 = (io_e == idx1).astype(jnp.float32)
    neg = jnp.where(oh1 > 0, -jnp.inf, logits)
    m2 = jnp.max(neg, axis=1, keepdims=True)
    idx2 = jnp.min(jnp.where(neg == m2, io_e, 1e9), axis=1, keepdims=True)
    oh2 = (io_e == idx2).astype(jnp.float32)

    e2 = jnp.exp(m2 - m1)
    p1 = 1.0 / (1.0 + e2)
    p2 = 1.0 - p1

    ex = jnp.exp(logits - m1)
    zden = jnp.sum(ex, axis=1, keepdims=True)
    probs = ex / zden
    pm_ref[...] += jnp.sum(probs, axis=0, keepdims=True)
    fr_ref[...] += jnp.sum(oh1, axis=0, keepdims=True)
    zlse = m1 + jnp.log(zden)
    z2_ref[...] += jnp.sum(zlse * zlse, axis=(0, 1), keepdims=True)

    s = oh1 + oh2
    row_i = lax.broadcasted_iota(jnp.int32, (_BT, _BT), 0)
    col_i = lax.broadcasted_iota(jnp.int32, (_BT, _BT), 1)
    tril = (row_i > col_i).astype(jnp.float32)
    prefix = lax.dot_general(tril, s, (((1,), (0,)), ((), ())),
                             preferred_element_type=jnp.float32)
    pb = prefix + base_ref[...]
    rank0 = jnp.sum(pb * oh1, axis=1, keepdims=True)
    rank1 = (jnp.sum(pb * oh2, axis=1, keepdims=True)
             + jnp.sum(oh1 * oh2, axis=1, keepdims=True))
    base_ref[...] += jnp.sum(s, axis=0, keepdims=True)

    keep0 = (rank0 < _CAP).astype(jnp.float32)
    keep1 = (rank1 < _CAP).astype(jnp.float32)
    slot0 = jnp.minimum(rank0, _CAP - 1.0)
    slot1 = jnp.minimum(rank1, _CAP - 1.0)
    tgt0 = jnp.sum(io_e * oh1, axis=1, keepdims=True)
    tgt1 = jnp.sum(io_e * oh2, axis=1, keepdims=True)
    gidx0 = tgt0 * _CAP + slot0
    gidx1 = tgt1 * _CAP + slot1

    n_io = lax.broadcasted_iota(jnp.int32, (_BT, 1), 0).astype(jnp.float32)
    dump = _EC + jnp.mod(n_io, 8.0)
    dst0_ref[...] = jnp.where(keep0 > 0, gidx0, dump).astype(jnp.int32)
    dst1_ref[...] = jnp.where(keep1 > 0, gidx1, dump).astype(jnp.int32)
    g0_ref[...] = gidx0.astype(jnp.int32)
    g1_ref[...] = gidx1.astype(jnp.int32)

    wa = p1 * keep0
    wb = p2 * keep1
    den = jnp.maximum(wa + wb, 1e-12)
    w0_ref[...] = wa / den
    w1_ref[...] = wb / den
    idx_ref[...] = jnp.concatenate([tgt0, tgt1], axis=1).astype(jnp.int32)

    pm = pm_ref[...] / _N
    fr = fr_ref[...] / _N
    bal = jnp.sum(pm * fr, axis=(0, 1), keepdims=True) * _E
    z2 = z2_ref[...] / _N
    aux_ref[...] = _AUX_W * bal + _Z_W * z2


def _router(x_flat, w_gate):
    f32 = jnp.float32
    i32 = jnp.int32
    return pl.pallas_call(
        _router_body,
        grid=(_NTB,),
        in_specs=[
            pl.BlockSpec((_BT, _D), lambda tb: (tb, 0)),
            pl.BlockSpec((_E, _D), lambda tb: (0, 0)),
        ],
        out_specs=[
            pl.BlockSpec((_BT, _K), lambda tb: (tb, 0)),
            pl.BlockSpec((_BT, 1), lambda tb: (tb, 0)),
            pl.BlockSpec((_BT, 1), lambda tb: (tb, 0)),
            pl.BlockSpec((_BT, 1), lambda tb: (tb, 0)),
            pl.BlockSpec((_BT, 1), lambda tb: (tb, 0)),
            pl.BlockSpec((_BT, 1), lambda tb: (tb, 0)),
            pl.BlockSpec((_BT, 1), lambda tb: (tb, 0)),
            pl.BlockSpec((1, 1), lambda tb: (0, 0)),
        ],
        out_shape=[
            jax.ShapeDtypeStruct((_N, _K), i32),
            jax.ShapeDtypeStruct((_N, 1), i32),
            jax.ShapeDtypeStruct((_N, 1), i32),
            jax.ShapeDtypeStruct((_N, 1), i32),
            jax.ShapeDtypeStruct((_N, 1), i32),
            jax.ShapeDtypeStruct((_N, 1), f32),
            jax.ShapeDtypeStruct((_N, 1), f32),
            jax.ShapeDtypeStruct((1, 1), f32),
        ],
        scratch_shapes=[
            pltpu.VMEM((1, _E), f32),
            pltpu.VMEM((1, _E), f32),
            pltpu.VMEM((1, _E), f32),
            pltpu.VMEM((1, 1), f32),
        ],
    )(x_flat, w_gate)


def _sc_scatter_body(x_hbm, idx_hbm, buf_hbm, idxg_v, src_v, rows_v, sem):
    w = lax.axis_index("s") * _SC_NC + lax.axis_index("c")
    base = w * (_SC_SCAT_CH * 64)
    pltpu.sync_copy(idx_hbm.at[w], idxg_v)
    for ch in range(_SC_SCAT_CH):
        cb = base + ch * 64
        for v in range(4):
            src_v[pl.ds(v * 16, 16)] = lax.rem(
                cb + v * 16 + lax.iota(jnp.int32, 16), _N)
        pltpu.async_copy(x_hbm.at[src_v], rows_v, sem).wait()
        pltpu.async_copy(rows_v, buf_hbm.at[idxg_v.at[ch]], sem).wait()


@functools.lru_cache(maxsize=None)
def _sc_scatter_kernel():
    return pl.kernel(
        _sc_scatter_body,
        out_type=jax.ShapeDtypeStruct((_EC + 8, _D), jnp.float32),
        mesh=plsc.VectorSubcoreMesh(core_axis_name="c", subcore_axis_name="s",
                                    num_cores=_SC_NC, num_subcores=_SC_NS),
        scratch_types=[
            pltpu.VMEM((_SC_SCAT_CH, 64), jnp.int32),
            pltpu.VMEM((64,), jnp.int32),
            pltpu.VMEM((64, _D), jnp.float32),
            pltpu.SemaphoreType.DMA,
        ],
    )


def _sc_scatter(x_flat, dst3):
    return _sc_scatter_kernel()(x_flat, dst3)


def _sc_gather_body(tab_hbm, idx_hbm, out_hbm, idx_v, rows_v, sem):
    w = lax.axis_index("s") * _SC_NC + lax.axis_index("c")
    base = w * _SC_GATH_ROWS
    pltpu.sync_copy(idx_hbm.at[pl.ds(base, _SC_GATH_ROWS)], idx_v)
    for ch in range(_SC_GATH_ROWS // 64):
        pltpu.async_copy(tab_hbm.at[idx_v.at[pl.ds(ch * 64, 64)]],
                         rows_v, sem).wait()
        pltpu.sync_copy(rows_v, out_hbm.at[pl.ds(base + ch * 64, 64)])


@functools.lru_cache(maxsize=None)
def _sc_gather_kernel():
    return pl.kernel(
        _sc_gather_body,
        out_type=jax.ShapeDtypeStruct((_NK, _D), jnp.float32),
        mesh=plsc.VectorSubcoreMesh(core_axis_name="c", subcore_axis_name="s",
                                    num_cores=_SC_NC, num_subcores=_SC_NS),
        scratch_types=[
            pltpu.VMEM((_SC_GATH_ROWS,), jnp.int32),
            pltpu.VMEM((64, _D), jnp.float32),
            pltpu.SemaphoreType.DMA,
        ],
    )


def _sc_gather(tab, gidx):
    return _sc_gather_kernel()(tab, gidx)


def _ffn_body(buf_ref, wg_ref, wu_ref, w2_ref, out_ref):
    hb = pl.program_id(1)
    bf16 = jnp.bfloat16
    xb = buf_ref[...].astype(bf16)
    g = jnp.dot(xb, wg_ref[0].astype(bf16), preferred_element_type=jnp.float32)
    u = jnp.dot(xb, wu_ref[0].astype(bf16), preferred_element_type=jnp.float32)
    act = g * (1.0 / (1.0 + jnp.exp(-g))) * u
    part = jnp.dot(act.astype(bf16), w2_ref[0].astype(bf16),
                   preferred_element_type=jnp.float32)

    @pl.when(hb == 0)
    def _set():
        out_ref[...] = part

    @pl.when(hb > 0)
    def _acc():
        out_ref[...] += part


def _ffn(buf, w13, w2):
    nhb = _H // _BH
    return pl.pallas_call(
        _ffn_body,
        grid=(_E, nhb),
        in_specs=[
            pl.BlockSpec((_CAP, _D), lambda e, h: (e, 0)),
            pl.BlockSpec((1, _D, _BH), lambda e, h: (e, 0, h)),
            pl.BlockSpec((1, _D, _BH), lambda e, h: (e, 0, nhb + h)),
            pl.BlockSpec((1, _BH, _D), lambda e, h: (e, h, 0)),
        ],
        out_specs=pl.BlockSpec((_CAP, _D), lambda e, h: (e, 0)),
        out_shape=jax.ShapeDtypeStruct((_EC, _D), jnp.float32),
        compiler_params=pltpu.CompilerParams(
            dimension_semantics=("arbitrary", "arbitrary")),
    )(buf, w13, w13, w2)


def _gelu(v):
    return 0.5 * v * (1.0 + lax.erf(v * 0.7071067811865476))


def _dot_rt(a, b):
    return lax.dot_general(a, b, (((1,), (1,)), ((), ())),
                           preferred_element_type=jnp.float32)


def _collab_body(s0_ref, s1_ref, w0_ref, w1_ref, wm_ref, wq_ref, wk_ref,
                 wu1_ref, wu2_ref, y_ref):
    w0 = w0_ref[...]
    w1 = w1_ref[...]
    km0 = (w0 > 0).astype(jnp.float32)
    km1 = (w1 > 0).astype(jnp.float32)
    s0 = jnp.where(w0 > 0, s0_ref[...], 0.0)
    s1 = jnp.where(w1 > 0, s1_ref[...], 0.0)

    bf16 = jnp.bfloat16
    wm = wm_ref[...].astype(bf16)
    wq = wq_ref[...].astype(bf16)
    wk = wk_ref[...].astype(bf16)
    s0h = s0.astype(bf16)
    s1h = s1.astype(bf16)
    m0 = _dot_rt(s0h, wm)
    m1 = _dot_rt(s1h, wm)
    q0 = _dot_rt(s0h, wq)
    q1 = _dot_rt(s1h, wq)
    k0 = _dot_rt(m0.astype(bf16), wk)
    k1 = _dot_rt(m1.astype(bf16), wk)

    inv = 1.0 / math.sqrt(_D)
    s00 = jnp.sum(q0 * k0, axis=1, keepdims=True) * inv
    s01 = jnp.sum(q0 * k1, axis=1, keepdims=True) * inv
    s10 = jnp.sum(q1 * k0, axis=1, keepdims=True) * inv
    s11 = jnp.sum(q1 * k1, axis=1, keepdims=True) * inv

    v00 = km0 * km0
    v01 = km0 * km1
    v10 = km1 * km0
    v11 = km1 * km1
    big = -1e9
    t00 = jnp.where(v00 > 0, s00, big)
    t01 = jnp.where(v01 > 0, s01, big)
    t10 = jnp.where(v10 > 0, s10, big)
    t11 = jnp.where(v11 > 0, s11, big)

    mx0 = jnp.maximum(t00, t01)
    p00 = jnp.exp(t00 - mx0) * v00
    p01 = jnp.exp(t01 - mx0) * v01
    z0 = jnp.maximum(p00 + p01, 1e-12)
    a00 = p00 / z0
    a01 = p01 / z0
    mx1 = jnp.maximum(t10, t11)
    p10 = jnp.exp(t10 - mx1) * v10
    p11 = jnp.exp(t11 - mx1) * v11
    z1 = jnp.maximum(p10 + p11, 1e-12)
    a10 = p10 / z1
    a11 = p11 / z1

    c0 = a00 * m0 + a01 * m1
    c1 = a10 * m0 + a11 * m1
    cat0 = jnp.concatenate([s0h, c0.astype(bf16)], axis=1)
    cat1 = jnp.concatenate([s1h, c1.astype(bf16)], axis=1)
    wu1 = wu1_ref[...].astype(bf16)
    wu2 = wu2_ref[...].astype(bf16)
    h0 = _gelu(_dot_rt(cat0, wu1))
    h1 = _gelu(_dot_rt(cat1, wu1))
    r0 = s0 + _dot_rt(h0.astype(bf16), wu2)
    r1 = s1 + _dot_rt(h1.astype(bf16), wu2)
    y_ref[...] = w0 * r0 + w1 * r1


def _collab(sel, w0, w1, wm, wq, wk, wu1, wu2):
    return pl.pallas_call(
        _collab_body,
        grid=(_NTB,),
        in_specs=[
            pl.BlockSpec((_BT, _D), lambda tb: (tb, 0)),
            pl.BlockSpec((_BT, _D), lambda tb: (_NTB + tb, 0)),
            pl.BlockSpec((_BT, 1), lambda tb: (tb, 0)),
            pl.BlockSpec((_BT, 1), lambda tb: (tb, 0)),
            pl.BlockSpec((_D, _D), lambda tb: (0, 0)),
            pl.BlockSpec((_D, _D), lambda tb: (0, 0)),
            pl.BlockSpec((_D, _D), lambda tb: (0, 0)),
            pl.BlockSpec((2 * _D, 2 * _D), lambda tb: (0, 0)),
            pl.BlockSpec((_D, 2 * _D), lambda tb: (0, 0)),
        ],
        out_specs=pl.BlockSpec((_BT, _D), lambda tb: (tb, 0)),
        out_shape=jax.ShapeDtypeStruct((_N, _D), jnp.float32),
    )(sel, sel, w0, w1, wm, wq, wk, wu1, wu2)


def kernel(x, W_gate, W13, W2, Wmsg, Wq, Wk, Wu1, Wu2):
    bx, tx, dx = x.shape
    x_flat = x.reshape(-1, dx).astype(jnp.float32)

    (topk_idx, dst0, dst1, g0, g1, w0, w1, aux) = _router(x_flat, W_gate)

    dst = jnp.concatenate([dst0[:, 0], dst1[:, 0]])
    buf = _sc_scatter(x_flat, dst.reshape(_NW, _SC_SCAT_CH, 64))
    out_buf = _ffn(buf, W13, W2)
    gidx = jnp.concatenate([g0[:, 0], g1[:, 0]])
    sel = _sc_gather(out_buf, gidx)

    y = _collab(sel, w0, w1, Wmsg, Wq, Wk, Wu1, Wu2)
    return (y.reshape(bx, tx, dx), aux[0, 0],
            topk_idx.reshape(bx, tx, _K))
